# baseline (speedup 1.0000x reference)
import functools

import jax
import jax.numpy as jnp
from jax import lax
from jax.experimental import pallas as pl
from jax.experimental.pallas import tpu as pltpu
from jax.experimental.pallas import tpu_sc as plsc

D = 128
B = 1024
L = 200
NCHUNK = 2
LC = L // NCHUNK
LCP = 104
NG = (LCP + 15) // 16
LPAD = NG * 16
NWORK = 32
RPW = B // NWORK


def _body(emb_hbm, idx_hbm, w0_hbm, ab0_hbm, w1_hbm, ab1_hbm, out_hbm,
          idx_v, emb_v, w_v, ab_v, out_v, sem0, sem1):
    wid = lax.axis_index("s") * 2 + lax.axis_index("c")
    base = wid * RPW
    pltpu.sync_copy(idx_hbm.at[pl.ds(base, RPW)], idx_v)
    pltpu.sync_copy(emb_hbm.at[pl.ds(base, RPW)], emb_v)

    tables = ((w0_hbm, ab0_hbm), (w1_hbm, ab1_hbm))
    sems = (sem0, sem1)

    def gather_copies(r, slot):
        cps = []
        for c, (w_hbm, ab_hbm) in enumerate(tables):
            cps.append(pltpu.make_async_copy(
                w_hbm.at[idx_v.at[r, c]],
                w_v.at[slot, pl.ds(c * LCP, LCP)], sems[slot]))
            cps.append(pltpu.make_async_copy(
                ab_hbm.at[idx_v.at[r, c]],
                ab_v.at[slot, pl.ds(c * LCP, LCP)], sems[slot]))
        return cps

    def fire(r, slot):
        for cp in gather_copies(r, slot):
            cp.start()

    def compute(r, slot):
        for c in range(NCHUNK):
            for g in range(NG):
                rowvec = (lax.broadcasted_iota(jnp.int32, (16,), 0)
                          + (c * LCP + g * 16))
                rowvec = jnp.minimum(rowvec, c * LCP + LCP - 1)

                def jbody(jb, accs):
                    accs = list(accs)
                    j0 = jb * 16
                    ev0 = emb_v[r, pl.ds(j0, 16)]
                    ev1 = emb_v[r, pl.ds(D + j0, 16)]
                    ev2 = emb_v[r, pl.ds(2 * D + j0, 16)]
                    jv = jnp.full((16,), j0, dtype=jnp.int32)
                    for jj in range(16):
                        p = jj % 4
                        wv = plsc.load_gather(w_v.at[slot], [rowvec, jv + jj])
                        accs[p] = accs[p] + wv * ev0[jj]
                        accs[4 + p] = accs[4 + p] + wv * ev1[jj]
                        accs[8 + p] = accs[8 + p] + wv * ev2[jj]
                    return tuple(accs)

                z = jnp.zeros((16,), jnp.float32)
                accs = lax.fori_loop(0, D // 16, jbody, (z,) * 12)
                a0 = (accs[0] + accs[1]) + (accs[2] + accs[3])
                a1 = (accs[4] + accs[5]) + (accs[6] + accs[7])
                a2 = (accs[8] + accs[9]) + (accs[10] + accs[11])

                def abcol(k):
                    return plsc.load_gather(
                        ab_v.at[slot],
                        [rowvec, jnp.full((16,), k, dtype=jnp.int32)])

                t0, t1, t2, tb = abcol(0), abcol(1), abcol(2), abcol(3)
                m = jnp.maximum(jnp.maximum(t0, t1), t2)
                x0 = jnp.exp(t0 - m)
                x1 = jnp.exp(t1 - m)
                x2 = jnp.exp(t2 - m)
                s = x0 + x1 + x2
                res = (x0 * a0 + x1 * a1 + x2 * a2) / s + tb
                out_v[r, c, pl.ds(g * 16, 16)] = res

    fire(0, 0)

    @pl.loop(0, RPW, step=2)
    def row_loop(rb):
        for b in range(2):
            r = rb + b

            @pl.when(r + 1 < RPW)
            def _():
                fire(r + 1, 1 - b)

            for cp in gather_copies(r, b):
                cp.wait()
            compute(r, b)

    pltpu.sync_copy(out_v, out_hbm.at[pl.ds(base, RPW)])


@jax.jit
def _sc_call(emb, idx, w0, ab0, w1, ab1):
    mesh = plsc.VectorSubcoreMesh(core_axis_name="c", subcore_axis_name="s",
                                  num_cores=2, num_subcores=16)
    fn = pl.kernel(
        _body,
        out_type=jax.ShapeDtypeStruct((B, NCHUNK, LPAD), jnp.float32),
        mesh=mesh,
        compiler_params=pltpu.CompilerParams(needs_layout_passes=False,
                                             use_tc_tiling_on_sc=False),
        scratch_types=[
            pltpu.VMEM((RPW, NCHUNK, LCP), jnp.int32),
            pltpu.VMEM((RPW, 3 * D), jnp.float32),
            pltpu.VMEM((2, NCHUNK * LCP, D), jnp.float32),
            pltpu.VMEM((2, NCHUNK * LCP, 16), jnp.float32),
            pltpu.VMEM((RPW, NCHUNK, LPAD), jnp.float32),
            pltpu.SemaphoreType.DMA,
            pltpu.SemaphoreType.DMA,
        ],
    )
    return fn(emb, idx, w0, ab0, w1, ab1)


def kernel(input_0, input_1, w0, b0, att0, w1, b1, att1):
    idx = input_1.astype(jnp.int32).reshape(B, NCHUNK, LC)
    idx = jnp.pad(idx, ((0, 0), (0, 0), (0, LCP - LC)))
    ab0 = jnp.pad(jnp.concatenate([att0, b0[:, None]], axis=1),
                  ((0, 0), (0, 12)))
    ab1 = jnp.pad(jnp.concatenate([att1, b1[:, None]], axis=1),
                  ((0, 0), (0, 12)))
    out = _sc_call(input_0, idx, w0, ab0, w1, ab1)
    return out[:, :, :LC].reshape(B, L)


# speedup vs baseline: 2.0398x; 2.0398x over previous
import functools

import jax
import jax.numpy as jnp
from jax import lax
from jax.experimental import pallas as pl
from jax.experimental.pallas import tpu as pltpu
from jax.experimental.pallas import tpu_sc as plsc

D = 128
B = 1024
L = 200
NCHUNK = 2
LC = L // NCHUNK
LCP = 104
NG = (LCP + 15) // 16
LPAD = NG * 16
NWORK = 32
RPW = B // NWORK


def _body(emb_hbm, idx_hbm, w0_hbm, ab0_hbm, w1_hbm, ab1_hbm, out_hbm,
          idx_v, emb_v, w_v, ab_v, out_v, sem0, sem1):
    wid = lax.axis_index("s") * 2 + lax.axis_index("c")
    base = wid * RPW
    pltpu.sync_copy(idx_hbm.at[pl.ds(base, RPW)], idx_v)
    pltpu.sync_copy(emb_hbm.at[pl.ds(base, RPW)], emb_v)

    tables = ((w0_hbm, ab0_hbm), (w1_hbm, ab1_hbm))
    sems = (sem0, sem1)

    def gather_copies(r, slot):
        cps = []
        for c, (w_hbm, ab_hbm) in enumerate(tables):
            cps.append(pltpu.make_async_copy(
                w_hbm.at[idx_v.at[r, c]],
                w_v.at[slot, pl.ds(c * LCP, LCP)], sems[slot]))
            cps.append(pltpu.make_async_copy(
                ab_hbm.at[idx_v.at[r, c]],
                ab_v.at[slot, pl.ds(c * LCP, LCP)], sems[slot]))
        return cps

    def fire(r, slot):
        for cp in gather_copies(r, slot):
            cp.start()

    lanes = lax.broadcasted_iota(jnp.int32, (16,), 0)

    def compute(r, slot):
        ev = [[emb_v[r, pl.ds(k * D + jb * 16, 16)] for jb in range(D // 16)]
              for k in range(3)]
        for c in range(NCHUNK):

            @pl.loop(0, NG)
            def group_loop(g):
                base = c * LCP + g * 16
                z = jnp.zeros((16,), jnp.float32)
                a0, a1, a2 = z, z, z
                for p in range(16):
                    row = base + p
                    s0, s1, s2 = z, z, z
                    for jb in range(D // 16):
                        wv = w_v[slot, row, pl.ds(jb * 16, 16)]
                        s0 = s0 + wv * ev[0][jb]
                        s1 = s1 + wv * ev[1][jb]
                        s2 = s2 + wv * ev[2][jb]
                    sel = lanes == p
                    a0 = jnp.where(sel, jnp.sum(s0), a0)
                    a1 = jnp.where(sel, jnp.sum(s1), a1)
                    a2 = jnp.where(sel, jnp.sum(s2), a2)

                rowvec = jnp.minimum(lanes + base, c * LCP + LCP - 1)

                def abcol(k):
                    return plsc.load_gather(
                        ab_v.at[slot],
                        [rowvec, jnp.full((16,), k, dtype=jnp.int32)])

                t0, t1, t2, tb = abcol(0), abcol(1), abcol(2), abcol(3)
                m = jnp.maximum(jnp.maximum(t0, t1), t2)
                x0 = jnp.exp(t0 - m)
                x1 = jnp.exp(t1 - m)
                x2 = jnp.exp(t2 - m)
                s = x0 + x1 + x2
                res = (x0 * a0 + x1 * a1 + x2 * a2) / s + tb
                out_v[r, c, pl.ds(g * 16, 16)] = res

    fire(0, 0)

    @pl.loop(0, RPW, step=2)
    def row_loop(rb):
        for b in range(2):
            r = rb + b

            @pl.when(r + 1 < RPW)
            def _():
                fire(r + 1, 1 - b)

            for cp in gather_copies(r, b):
                cp.wait()
            compute(r, b)

    pltpu.sync_copy(out_v, out_hbm.at[pl.ds(base, RPW)])


@jax.jit
def _sc_call(emb, idx, w0, ab0, w1, ab1):
    mesh = plsc.VectorSubcoreMesh(core_axis_name="c", subcore_axis_name="s",
                                  num_cores=2, num_subcores=16)
    fn = pl.kernel(
        _body,
        out_type=jax.ShapeDtypeStruct((B, NCHUNK, LPAD), jnp.float32),
        mesh=mesh,
        compiler_params=pltpu.CompilerParams(needs_layout_passes=False,
                                             use_tc_tiling_on_sc=False),
        scratch_types=[
            pltpu.VMEM((RPW, NCHUNK, LCP), jnp.int32),
            pltpu.VMEM((RPW, 3 * D), jnp.float32),
            pltpu.VMEM((2, NCHUNK * LCP, D), jnp.float32),
            pltpu.VMEM((2, NCHUNK * LCP, 16), jnp.float32),
            pltpu.VMEM((RPW, NCHUNK, LPAD), jnp.float32),
            pltpu.SemaphoreType.DMA,
            pltpu.SemaphoreType.DMA,
        ],
    )
    return fn(emb, idx, w0, ab0, w1, ab1)


def kernel(input_0, input_1, w0, b0, att0, w1, b1, att1):
    idx = input_1.astype(jnp.int32).reshape(B, NCHUNK, LC)
    idx = jnp.pad(idx, ((0, 0), (0, 0), (0, LCP - LC)))
    ab0 = jnp.pad(jnp.concatenate([att0, b0[:, None]], axis=1),
                  ((0, 0), (0, 12)))
    ab1 = jnp.pad(jnp.concatenate([att1, b1[:, None]], axis=1),
                  ((0, 0), (0, 12)))
    out = _sc_call(input_0, idx, w0, ab0, w1, ab1)
    return out[:, :, :LC].reshape(B, L)
